# Initial kernel scaffold; baseline (speedup 1.0000x reference)
import functools

import jax
import jax.numpy as jnp
import numpy as np
from jax import lax
from jax.experimental import pallas as pl
from jax.experimental.pallas import tpu as pltpu
from jax.experimental.pallas import tpu_sc as plsc

ALPHA = 0.05
GAMMA = 0.9
C1 = 1.0 - ALPHA
C2 = 1.0 - ALPHA + ALPHA * GAMMA
GOV = ALPHA * GAMMA / C1

L = 16
B = 64
T = 2048
NWORK = 32
EPW = B // NWORK

_k = np.arange(L, dtype=np.float64)
_CN1 = (C1 ** -_k).astype(np.float32)
_CP1 = (C1 ** _k).astype(np.float32)
_CS1 = (C1 ** (_k + 1)).astype(np.float32)
_CN2 = (C2 ** -_k).astype(np.float32)
_CP2 = (C2 ** _k).astype(np.float32)
_CS2 = (C2 ** (_k + 1)).astype(np.float32)
_E15 = np.eye(L, dtype=np.float32)[L - 1]


def _presence_body(la_ref, f0_ref, f1_ref):
    la0 = la_ref[0]
    la1 = la_ref[1]
    p0 = jnp.any(la0 >= la1)
    p1 = jnp.any(la1 > la0)
    ones = jnp.ones((8, 128), jnp.float32)
    zero = jnp.zeros((8, 128), jnp.float32)
    f0_ref[...] = jnp.where(p0, ones, zero)
    f1_ref[...] = jnp.where(p1, ones, zero)


def _scan_body(r0_hbm, r1_hbm, f0_hbm, f1_hbm, q0_hbm, q1_hbm,
               r0_v, r1_v, q0_v, q1_v, f_v):
    cid = lax.axis_index("c")
    sid = lax.axis_index("s")
    wid = sid * 2 + cid

    pltpu.sync_copy(f0_hbm.at[0], f_v)
    flag0 = f_v[pl.ds(0, L)] > 0.5
    pltpu.sync_copy(f1_hbm.at[0], f_v)
    flag1 = f_v[pl.ds(0, L)] > 0.5

    cn1 = jnp.asarray(_CN1)
    cp1 = jnp.asarray(_CP1)
    cs1 = jnp.asarray(_CS1)
    cn2 = jnp.asarray(_CN2)
    cp2 = jnp.asarray(_CP2)
    cs2 = jnp.asarray(_CS2)
    e15 = jnp.asarray(_E15)

    for ei in range(EPW):
        ep = wid * EPW + ei
        pltpu.sync_copy(r0_hbm.at[ep], r0_v)
        pltpu.sync_copy(r1_hbm.at[ep], r1_v)

        def chunk(j, carry):
            dc, sc = carry
            b0 = r0_v[pl.ds(j * L, L)]
            b1 = r1_v[pl.ds(j * L, L)]
            r20 = jnp.where(flag0, 2.0 * b0 - 1.0, b0)
            r21 = jnp.where(flag1, 2.0 * b1 - 1.0, b1)
            bu = ALPHA * (r20 - r21)
            bv = ALPHA * (r20 + r21)
            dch = plsc.cumsum(bu * cn1) * cp1 + dc * cs1
            bw = bv + GOV * jnp.abs(dch - bu)
            sch = plsc.cumsum(bw * cn2) * cp2 + sc * cs2
            q0_v[pl.ds(j * L, L)] = 0.5 * (sch + dch)
            q1_v[pl.ds(j * L, L)] = 0.5 * (sch - dch)
            return jnp.sum(dch * e15), jnp.sum(sch * e15)

        lax.fori_loop(0, T // L, chunk,
                      (jnp.float32(0.0), jnp.float32(1.0)))
        pltpu.sync_copy(q0_v, q0_hbm.at[ep])
        pltpu.sync_copy(q1_v, q1_hbm.at[ep])


_sc_scan = functools.partial(
    pl.kernel,
    out_type=(jax.ShapeDtypeStruct((B, T), jnp.float32),
              jax.ShapeDtypeStruct((B, T), jnp.float32)),
    mesh=plsc.VectorSubcoreMesh(core_axis_name="c", subcore_axis_name="s"),
    scratch_types=[
        pltpu.VMEM((T,), jnp.float32),
        pltpu.VMEM((T,), jnp.float32),
        pltpu.VMEM((T,), jnp.float32),
        pltpu.VMEM((T,), jnp.float32),
        pltpu.VMEM((128,), jnp.float32),
    ],
)(_scan_body)


def kernel(state, last_action, rewards):
    del state
    la_t = jnp.moveaxis(last_action, -1, 0)
    f0, f1 = pl.pallas_call(
        _presence_body,
        out_shape=(jax.ShapeDtypeStruct((8, 128), jnp.float32),
                   jax.ShapeDtypeStruct((8, 128), jnp.float32)),
    )(la_t)
    r0 = rewards[..., 0]
    r1 = rewards[..., 1]
    q0, q1 = _sc_scan(r0, r1, f0, f1)
    return jnp.stack([q0, q1], axis=-1)


# speedup vs baseline: 4.0801x; 4.0801x over previous
import functools

import jax
import jax.numpy as jnp
import numpy as np
from jax import lax
from jax.experimental import pallas as pl
from jax.experimental.pallas import tpu as pltpu
from jax.experimental.pallas import tpu_sc as plsc

ALPHA = 0.05
GAMMA = 0.9
C1 = 1.0 - ALPHA
C2 = 1.0 - ALPHA + ALPHA * GAMMA
GOV = ALPHA * GAMMA / C1

L = 16
B = 64
T = 2048
NWORK = 32
EPW = B // NWORK

_LN1 = float(np.log(C1))
_LN2 = float(np.log(C2))


def _presence_body(la_ref, f0_ref, f1_ref):
    la0 = la_ref[0]
    la1 = la_ref[1]
    p0 = jnp.any(la0 >= la1)
    p1 = jnp.any(la1 > la0)
    ones = jnp.ones((8, 128), jnp.float32)
    zero = jnp.zeros((8, 128), jnp.float32)
    f0_ref[...] = jnp.where(p0, ones, zero)
    f1_ref[...] = jnp.where(p1, ones, zero)


def _scan_body(r0_hbm, r1_hbm, f0_hbm, f1_hbm, q0_hbm, q1_hbm,
               r0_v, r1_v, q0_v, q1_v, f_v):
    cid = lax.axis_index("c")
    sid = lax.axis_index("s")
    wid = sid * 2 + cid

    pltpu.sync_copy(f0_hbm.at[0], f_v)
    flag0 = f_v[pl.ds(0, L)] > 0.5
    pltpu.sync_copy(f1_hbm.at[0], f_v)
    flag1 = f_v[pl.ds(0, L)] > 0.5

    kf = lax.iota(jnp.int32, L).astype(jnp.float32)
    cn1 = jnp.exp(kf * jnp.float32(-_LN1))
    cp1 = jnp.exp(kf * jnp.float32(_LN1))
    cs1 = cp1 * jnp.float32(C1)
    cn2 = jnp.exp(kf * jnp.float32(-_LN2))
    cp2 = jnp.exp(kf * jnp.float32(_LN2))
    cs2 = cp2 * jnp.float32(C2)
    lane15 = lax.iota(jnp.int32, L) == (L - 1)

    for ei in range(EPW):
        ep = wid * EPW + ei
        pltpu.sync_copy(r0_hbm.at[ep], r0_v)
        pltpu.sync_copy(r1_hbm.at[ep], r1_v)

        def chunk(j, carry):
            dc, sc = carry
            b0 = r0_v[pl.ds(j * L, L)]
            b1 = r1_v[pl.ds(j * L, L)]
            r20 = jnp.where(flag0, 2.0 * b0 - 1.0, b0)
            r21 = jnp.where(flag1, 2.0 * b1 - 1.0, b1)
            bu = ALPHA * (r20 - r21)
            bv = ALPHA * (r20 + r21)
            dch = plsc.cumsum(bu * cn1) * cp1 + dc * cs1
            bw = bv + GOV * jnp.abs(dch - bu)
            sch = plsc.cumsum(bw * cn2) * cp2 + sc * cs2
            q0_v[pl.ds(j * L, L)] = 0.5 * (sch + dch)
            q1_v[pl.ds(j * L, L)] = 0.5 * (sch - dch)
            zero = jnp.zeros((L,), jnp.float32)
            return (jnp.sum(jnp.where(lane15, dch, zero)),
                    jnp.sum(jnp.where(lane15, sch, zero)))

        lax.fori_loop(0, T // L, chunk,
                      (jnp.float32(0.0), jnp.float32(1.0)))
        pltpu.sync_copy(q0_v, q0_hbm.at[ep])
        pltpu.sync_copy(q1_v, q1_hbm.at[ep])


_sc_scan = functools.partial(
    pl.kernel,
    out_type=(jax.ShapeDtypeStruct((B, T), jnp.float32),
              jax.ShapeDtypeStruct((B, T), jnp.float32)),
    mesh=plsc.VectorSubcoreMesh(core_axis_name="c", subcore_axis_name="s",
                                num_cores=2, num_subcores=16),
    scratch_types=[
        pltpu.VMEM((T,), jnp.float32),
        pltpu.VMEM((T,), jnp.float32),
        pltpu.VMEM((T,), jnp.float32),
        pltpu.VMEM((T,), jnp.float32),
        pltpu.VMEM((128,), jnp.float32),
    ],
    compiler_params=pltpu.CompilerParams(needs_layout_passes=False),
)(_scan_body)


def kernel(state, last_action, rewards):
    del state
    la_t = jnp.moveaxis(last_action, -1, 0)
    f0, f1 = pl.pallas_call(
        _presence_body,
        out_shape=(jax.ShapeDtypeStruct((8, 128), jnp.float32),
                   jax.ShapeDtypeStruct((8, 128), jnp.float32)),
    )(la_t)
    r0 = rewards[..., 0]
    r1 = rewards[..., 1]
    q0, q1 = _sc_scan(r0, r1, f0, f1)
    return jnp.stack([q0, q1], axis=-1)
